# Initial kernel scaffold; baseline (speedup 1.0000x reference)
import functools

import jax
import jax.numpy as jnp
from jax import lax
from jax.experimental import pallas as pl
from jax.experimental.pallas import tpu as pltpu
from jax.experimental.pallas import tpu_sc as plsc

N = 10000
D = 128
E = 320000
EPS = 1e-5
NP = 10240
NC = 2
NS = 16
NW = NC * NS
EPT = E // NW
CH = 128
NFULL = EPT // CH
REM = EPT - NFULL * CH
RPT = N // NS
DPT = NP // NS

_mesh = plsc.VectorSubcoreMesh(core_axis_name="c", subcore_axis_name="s")


@functools.partial(
    pl.kernel,
    out_type=jax.ShapeDtypeStruct((NC * NP,), jnp.float32),
    mesh=_mesh,
    scratch_types=[
        pltpu.VMEM((EPT,), jnp.int32),
        pltpu.VMEM((CH,), jnp.int32),
        pltpu.VMEM((REM,), jnp.int32),
        pltpu.VMEM((CH,), jnp.float32),
        pltpu.VMEM((DPT,), jnp.float32),
        pltpu.VMEM_SHARED((NP,), jnp.float32),
    ],
)
def _deg_kernel(dst_hbm, out_hbm, dst_v, didx, didx_r, ones, zbuf, dacc):
    c = lax.axis_index("c")
    s = lax.axis_index("s")
    wid = s * NC + c

    def fill_ones(i, carry):
        ones[pl.ds(i * 16, 16)] = jnp.full((16,), 1.0, jnp.float32)
        return carry

    lax.fori_loop(0, CH // 16, fill_ones, 0)

    def fill_z(i, carry):
        zbuf[pl.ds(i * 16, 16)] = jnp.zeros((16,), jnp.float32)
        return carry

    lax.fori_loop(0, DPT // 16, fill_z, 0)

    pltpu.sync_copy(zbuf, dacc.at[pl.ds(s * DPT, DPT)])
    pltpu.sync_copy(dst_hbm.at[pl.ds(wid * EPT, EPT)], dst_v)
    plsc.subcore_barrier()

    def body(j, carry):
        pltpu.sync_copy(dst_v.at[pl.ds(j * CH, CH)], didx)
        pltpu.sync_copy(ones, dacc.at[didx], add=True)
        return carry

    lax.fori_loop(0, NFULL, body, 0)
    pltpu.sync_copy(dst_v.at[pl.ds(NFULL * CH, REM)], didx_r)
    pltpu.sync_copy(ones.at[pl.ds(0, REM)], dacc.at[didx_r], add=True)
    plsc.subcore_barrier()
    pltpu.sync_copy(dacc.at[pl.ds(s * DPT, DPT)],
                    out_hbm.at[pl.ds(c * NP + s * DPT, DPT)])


@functools.partial(
    pl.kernel,
    out_type=jax.ShapeDtypeStruct((NC * N, D), jnp.float32),
    mesh=_mesh,
    scratch_types=[
        pltpu.VMEM((EPT,), jnp.int32),
        pltpu.VMEM((EPT,), jnp.int32),
        pltpu.VMEM((CH,), jnp.int32),
        pltpu.VMEM((CH,), jnp.int32),
        pltpu.VMEM((REM,), jnp.int32),
        pltpu.VMEM((REM,), jnp.int32),
        pltpu.VMEM((CH, D), jnp.float32),
        pltpu.VMEM((REM, D), jnp.float32),
        pltpu.VMEM_SHARED((N, D), jnp.float32),
        pltpu.SemaphoreType.DMA,
    ],
)
def _scatter_kernel(hs_hbm, src_hbm, dst_hbm, zeros_hbm, out_hbm,
                    src_v, dst_v, sidx, didx, sidx_r, didx_r,
                    gbuf, gbuf_r, acc, sem):
    c = lax.axis_index("c")
    s = lax.axis_index("s")
    wid = s * NC + c

    pltpu.sync_copy(src_hbm.at[pl.ds(wid * EPT, EPT)], src_v)
    pltpu.sync_copy(dst_hbm.at[pl.ds(wid * EPT, EPT)], dst_v)

    rows = pl.ds(s * RPT, RPT)

    @pl.when(c == 0)
    def _():
        pltpu.sync_copy(hs_hbm.at[rows], acc.at[rows])

    @pl.when(c == 1)
    def _():
        pltpu.sync_copy(zeros_hbm.at[rows], acc.at[rows])

    plsc.subcore_barrier()

    def body(j, carry):
        pltpu.sync_copy(src_v.at[pl.ds(j * CH, CH)], sidx)
        pltpu.sync_copy(dst_v.at[pl.ds(j * CH, CH)], didx)
        pltpu.async_copy(hs_hbm.at[sidx], gbuf, sem).wait()
        pltpu.sync_copy(gbuf, acc.at[didx], add=True)
        return carry

    lax.fori_loop(0, NFULL, body, 0)

    pltpu.sync_copy(src_v.at[pl.ds(NFULL * CH, REM)], sidx_r)
    pltpu.sync_copy(dst_v.at[pl.ds(NFULL * CH, REM)], didx_r)
    pltpu.async_copy(hs_hbm.at[sidx_r], gbuf_r, sem).wait()
    pltpu.sync_copy(gbuf_r, acc.at[didx_r], add=True)

    plsc.subcore_barrier()
    pltpu.sync_copy(acc.at[rows], out_hbm.at[pl.ds(c * N + s * RPT, RPT)])


def _dinv_col(deg_ref):
    dsum = deg_ref[0:1, :] + deg_ref[1:2, :] + 1.0
    return jnp.transpose(lax.rsqrt(dsum))[:N, :]


def _mm1_body(x_ref, w_ref, deg_ref, o_ref):
    dcol = _dinv_col(deg_ref)
    xs = x_ref[...] * dcol
    o_ref[...] = jnp.dot(xs, w_ref[...], preferred_element_type=jnp.float32)


def _bn(agg, g_ref, be_ref):
    mu = jnp.mean(agg, axis=0, keepdims=True)
    var = jnp.mean(agg * agg, axis=0, keepdims=True) - mu * mu
    return (agg - mu) * lax.rsqrt(var + EPS) * g_ref[...] + be_ref[...]


def _post1_body(acc_ref, deg_ref, b_ref, g_ref, be_ref, w2_ref, o_ref):
    dcol = _dinv_col(deg_ref)
    agg = (acc_ref[0:N, :] + acc_ref[N:2 * N, :]) * dcol + b_ref[...]
    y = jnp.maximum(_bn(agg, g_ref, be_ref), 0.0)
    h2 = jnp.dot(y, w2_ref[...], preferred_element_type=jnp.float32)
    o_ref[...] = h2 * dcol


def _post2_body(acc_ref, deg_ref, b_ref, g_ref, be_ref, o_ref):
    dcol = _dinv_col(deg_ref)
    agg = (acc_ref[0:N, :] + acc_ref[N:2 * N, :]) * dcol + b_ref[...]
    o_ref[...] = _bn(agg, g_ref, be_ref)


def kernel(x, edge_index, W1, b1, gamma1, beta1, W2, b2, gamma2, beta2):
    src = edge_index[0]
    dst = edge_index[1]
    zeros = jnp.zeros((N, D), jnp.float32)

    deg2 = _deg_kernel(dst).reshape(NC, NP)

    hs1 = pl.pallas_call(
        _mm1_body,
        out_shape=jax.ShapeDtypeStruct((N, D), jnp.float32),
    )(x, W1, deg2)

    acc1 = _scatter_kernel(hs1, src, dst, zeros)

    hs2 = pl.pallas_call(
        _post1_body,
        out_shape=jax.ShapeDtypeStruct((N, D), jnp.float32),
    )(acc1, deg2, b1.reshape(1, D), gamma1.reshape(1, D),
      beta1.reshape(1, D), W2)

    acc2 = _scatter_kernel(hs2, src, dst, zeros)

    out = pl.pallas_call(
        _post2_body,
        out_shape=jax.ShapeDtypeStruct((N, D), jnp.float32),
    )(acc2, deg2, b2.reshape(1, D), gamma2.reshape(1, D),
      beta2.reshape(1, D))
    return out


# speedup vs baseline: 23.4172x; 23.4172x over previous
import functools

import jax
import jax.numpy as jnp
from jax import lax
from jax.experimental import pallas as pl
from jax.experimental.pallas import tpu as pltpu
from jax.experimental.pallas import tpu_sc as plsc

N = 10000
D = 128
E = 320000
EPS = 1e-5
NP = 10240
NC = 2
NS = 16
NW = NC * NS
CH = 128
RT = 79
E_PAD = NW * RT * CH
PAD = E_PAD - E
NH = 10112
RPT = NH // NS
DPT = NP // NS

_mesh = plsc.VectorSubcoreMesh(core_axis_name="c", subcore_axis_name="s")


@functools.partial(
    pl.kernel,
    out_type=jax.ShapeDtypeStruct((NC * NP,), jnp.float32),
    mesh=_mesh,
    scratch_types=[
        pltpu.VMEM((RT, CH), jnp.int32),
        pltpu.VMEM((CH,), jnp.float32),
        pltpu.VMEM((DPT,), jnp.float32),
        pltpu.VMEM_SHARED((NP,), jnp.float32),
    ],
)
def _deg_kernel(dst_hbm, out_hbm, dst_v, ones, zbuf, dacc):
    c = lax.axis_index("c")
    s = lax.axis_index("s")
    wid = s * NC + c

    def fill_ones(i, carry):
        ones[pl.ds(i * 16, 16)] = jnp.full((16,), 1.0, jnp.float32)
        return carry

    lax.fori_loop(0, CH // 16, fill_ones, 0)

    def fill_z(i, carry):
        zbuf[pl.ds(i * 16, 16)] = jnp.zeros((16,), jnp.float32)
        return carry

    lax.fori_loop(0, DPT // 16, fill_z, 0)

    pltpu.sync_copy(zbuf, dacc.at[pl.ds(s * DPT, DPT)])
    pltpu.sync_copy(dst_hbm.at[wid], dst_v)
    plsc.subcore_barrier()

    def body(j, carry):
        pltpu.sync_copy(ones, dacc.at[dst_v.at[j]], add=True)
        return carry

    lax.fori_loop(0, RT, body, 0)
    plsc.subcore_barrier()
    pltpu.sync_copy(dacc.at[pl.ds(s * DPT, DPT)],
                    out_hbm.at[pl.ds(c * NP + s * DPT, DPT)])


@functools.partial(
    pl.kernel,
    out_type=jax.ShapeDtypeStruct((NC * NH, D), jnp.float32),
    mesh=_mesh,
    scratch_types=[
        pltpu.VMEM((RT, CH), jnp.int32),
        pltpu.VMEM((RT, CH), jnp.int32),
        pltpu.VMEM((CH, D), jnp.float32),
        pltpu.VMEM_SHARED((NH, D), jnp.float32),
        pltpu.SemaphoreType.DMA,
    ],
)
def _scatter_kernel(hs_hbm, src_hbm, dst_hbm, zeros_hbm, out_hbm,
                    src_v, dst_v, gbuf, acc, sem):
    c = lax.axis_index("c")
    s = lax.axis_index("s")
    wid = s * NC + c

    pltpu.sync_copy(src_hbm.at[wid], src_v)
    pltpu.sync_copy(dst_hbm.at[wid], dst_v)

    rows = pl.ds(s * RPT, RPT)

    @pl.when(c == 0)
    def _():
        pltpu.sync_copy(hs_hbm.at[rows], acc.at[rows])

    @pl.when(c == 1)
    def _():
        pltpu.sync_copy(zeros_hbm.at[rows], acc.at[rows])

    plsc.subcore_barrier()

    def body(j, carry):
        pltpu.async_copy(hs_hbm.at[src_v.at[j]], gbuf, sem).wait()
        pltpu.sync_copy(gbuf, acc.at[dst_v.at[j]], add=True)
        return carry

    lax.fori_loop(0, RT, body, 0)

    plsc.subcore_barrier()
    pltpu.sync_copy(acc.at[rows], out_hbm.at[pl.ds(c * NH + s * RPT, RPT)])


def _dinv_col(deg_ref):
    dsum = deg_ref[0:1, :] + deg_ref[1:2, :] + 1.0
    return jnp.transpose(lax.rsqrt(dsum))[:N, :]


def _mm1_body(x_ref, w_ref, deg_ref, o_ref):
    dcol = _dinv_col(deg_ref)
    xs = x_ref[...] * dcol
    o_ref[0:N, :] = jnp.dot(xs, w_ref[...], preferred_element_type=jnp.float32)
    o_ref[N:NH, :] = jnp.zeros((NH - N, D), jnp.float32)


def _bn(agg, g_ref, be_ref):
    mu = jnp.mean(agg, axis=0, keepdims=True)
    var = jnp.mean(agg * agg, axis=0, keepdims=True) - mu * mu
    return (agg - mu) * lax.rsqrt(var + EPS) * g_ref[...] + be_ref[...]


def _post1_body(acc_ref, deg_ref, b_ref, g_ref, be_ref, w2_ref, o_ref):
    dcol = _dinv_col(deg_ref)
    agg = (acc_ref[0:N, :] + acc_ref[NH:NH + N, :]) * dcol + b_ref[...]
    y = jnp.maximum(_bn(agg, g_ref, be_ref), 0.0)
    h2 = jnp.dot(y, w2_ref[...], preferred_element_type=jnp.float32)
    o_ref[0:N, :] = h2 * dcol
    o_ref[N:NH, :] = jnp.zeros((NH - N, D), jnp.float32)


def _post2_body(acc_ref, deg_ref, b_ref, g_ref, be_ref, o_ref):
    dcol = _dinv_col(deg_ref)
    agg = (acc_ref[0:N, :] + acc_ref[NH:NH + N, :]) * dcol + b_ref[...]
    o_ref[...] = _bn(agg, g_ref, be_ref)


def kernel(x, edge_index, W1, b1, gamma1, beta1, W2, b2, gamma2, beta2):
    pad_iota = jnp.arange(PAD, dtype=jnp.int32)
    src2d = jnp.concatenate(
        [edge_index[0], pad_iota % N]).reshape(NW, RT, CH)
    dst2d = jnp.concatenate(
        [edge_index[1], N + (pad_iota % 8)]).reshape(NW, RT, CH)
    zeros = jnp.zeros((NH, D), jnp.float32)

    deg2 = _deg_kernel(dst2d).reshape(NC, NP)

    hs1 = pl.pallas_call(
        _mm1_body,
        out_shape=jax.ShapeDtypeStruct((NH, D), jnp.float32),
    )(x, W1, deg2)

    acc1 = _scatter_kernel(hs1, src2d, dst2d, zeros)

    hs2 = pl.pallas_call(
        _post1_body,
        out_shape=jax.ShapeDtypeStruct((NH, D), jnp.float32),
    )(acc1, deg2, b1.reshape(1, D), gamma1.reshape(1, D),
      beta1.reshape(1, D), W2)

    acc2 = _scatter_kernel(hs2, src2d, dst2d, zeros)

    out = pl.pallas_call(
        _post2_body,
        out_shape=jax.ShapeDtypeStruct((N, D), jnp.float32),
    )(acc2, deg2, b2.reshape(1, D), gamma2.reshape(1, D),
      beta2.reshape(1, D))
    return out


# speedup vs baseline: 25.3186x; 1.0812x over previous
import functools

import jax
import jax.numpy as jnp
from jax import lax
from jax.experimental import pallas as pl
from jax.experimental.pallas import tpu as pltpu
from jax.experimental.pallas import tpu_sc as plsc

N = 10000
D = 128
E = 320000
EPS = 1e-5
NP = 10240
NC = 2
NS = 16
NW = NC * NS
CH = 128
RT = 80
E_PAD = NW * RT * CH
PAD = E_PAD - E
NH = 10112
RPT = NH // NS
DPT = NP // NS

_mesh = plsc.VectorSubcoreMesh(core_axis_name="c", subcore_axis_name="s")


@functools.partial(
    pl.kernel,
    out_type=jax.ShapeDtypeStruct((NC * NP,), jnp.float32),
    mesh=_mesh,
    scratch_types=[
        pltpu.VMEM((RT, CH), jnp.int32),
        pltpu.VMEM((CH,), jnp.float32),
        pltpu.VMEM((DPT,), jnp.float32),
        pltpu.VMEM_SHARED((NP,), jnp.float32),
    ],
)
def _deg_kernel(dst_hbm, out_hbm, dst_v, ones, zbuf, dacc):
    c = lax.axis_index("c")
    s = lax.axis_index("s")
    wid = s * NC + c

    def fill_ones(i, carry):
        ones[pl.ds(i * 16, 16)] = jnp.full((16,), 1.0, jnp.float32)
        return carry

    lax.fori_loop(0, CH // 16, fill_ones, 0)

    def fill_z(i, carry):
        zbuf[pl.ds(i * 16, 16)] = jnp.zeros((16,), jnp.float32)
        return carry

    lax.fori_loop(0, DPT // 16, fill_z, 0)

    pltpu.sync_copy(zbuf, dacc.at[pl.ds(s * DPT, DPT)])
    pltpu.sync_copy(dst_hbm.at[wid], dst_v)
    plsc.subcore_barrier()

    def body(j, carry):
        pltpu.sync_copy(ones, dacc.at[dst_v.at[j]], add=True)
        return carry

    lax.fori_loop(0, RT, body, 0)
    plsc.subcore_barrier()
    pltpu.sync_copy(dacc.at[pl.ds(s * DPT, DPT)],
                    out_hbm.at[pl.ds(c * NP + s * DPT, DPT)])


@functools.partial(
    pl.kernel,
    out_type=jax.ShapeDtypeStruct((NC * NH, D), jnp.float32),
    mesh=_mesh,
    scratch_types=[
        pltpu.VMEM((RT, CH), jnp.int32),
        pltpu.VMEM((RT, CH), jnp.int32),
        pltpu.VMEM((CH // 2, D), jnp.float32),
        pltpu.VMEM((CH // 2, D), jnp.float32),
        pltpu.VMEM_SHARED((NH, D), jnp.float32),
        pltpu.SemaphoreType.DMA,
    ],
)
def _scatter_kernel(hs_hbm, src_hbm, dst_hbm, zeros_hbm, out_hbm,
                    src_v, dst_v, g0, g1, acc, sem0):
    c = lax.axis_index("c")
    s = lax.axis_index("s")
    wid = s * NC + c

    pltpu.sync_copy(src_hbm.at[wid], src_v)
    pltpu.sync_copy(dst_hbm.at[wid], dst_v)

    rows = pl.ds(s * RPT, RPT)

    @pl.when(c == 0)
    def _():
        pltpu.sync_copy(hs_hbm.at[rows], acc.at[rows])

    @pl.when(c == 1)
    def _():
        pltpu.sync_copy(zeros_hbm.at[rows], acc.at[rows])

    plsc.subcore_barrier()

    def body(j, carry):
        ha = pltpu.async_copy(hs_hbm.at[src_v.at[j, pl.ds(0, 64)]],
                              g0, sem0)
        hb = pltpu.async_copy(hs_hbm.at[src_v.at[j, pl.ds(64, 64)]],
                              g1, sem0)
        ha.wait()
        pltpu.sync_copy(g0, acc.at[dst_v.at[j, pl.ds(0, 64)]], add=True)
        hb.wait()
        pltpu.sync_copy(g1, acc.at[dst_v.at[j, pl.ds(64, 64)]], add=True)
        return carry

    lax.fori_loop(0, RT, body, 0)

    plsc.subcore_barrier()
    pltpu.sync_copy(acc.at[rows], out_hbm.at[pl.ds(c * NH + s * RPT, RPT)])


def _dinv_col(deg_ref):
    dsum = deg_ref[0:1, :] + deg_ref[1:2, :] + 1.0
    return jnp.transpose(lax.rsqrt(dsum))[:N, :]


def _mm1_body(x_ref, w_ref, deg_ref, o_ref):
    dcol = _dinv_col(deg_ref)
    xs = x_ref[...] * dcol
    o_ref[0:N, :] = jnp.dot(xs, w_ref[...], preferred_element_type=jnp.float32)
    o_ref[N:NH, :] = jnp.zeros((NH - N, D), jnp.float32)


def _bn(agg, g_ref, be_ref):
    mu = jnp.mean(agg, axis=0, keepdims=True)
    var = jnp.mean(agg * agg, axis=0, keepdims=True) - mu * mu
    return (agg - mu) * lax.rsqrt(var + EPS) * g_ref[...] + be_ref[...]


def _post1_body(acc_ref, deg_ref, b_ref, g_ref, be_ref, w2_ref, o_ref):
    dcol = _dinv_col(deg_ref)
    agg = (acc_ref[0:N, :] + acc_ref[NH:NH + N, :]) * dcol + b_ref[...]
    y = jnp.maximum(_bn(agg, g_ref, be_ref), 0.0)
    h2 = jnp.dot(y, w2_ref[...], preferred_element_type=jnp.float32)
    o_ref[0:N, :] = h2 * dcol
    o_ref[N:NH, :] = jnp.zeros((NH - N, D), jnp.float32)


def _post2_body(acc_ref, deg_ref, b_ref, g_ref, be_ref, o_ref):
    dcol = _dinv_col(deg_ref)
    agg = (acc_ref[0:N, :] + acc_ref[NH:NH + N, :]) * dcol + b_ref[...]
    o_ref[...] = _bn(agg, g_ref, be_ref)


def kernel(x, edge_index, W1, b1, gamma1, beta1, W2, b2, gamma2, beta2):
    pad_iota = jnp.arange(PAD, dtype=jnp.int32)
    src2d = jnp.concatenate(
        [edge_index[0], pad_iota % N]).reshape(NW, RT, CH)
    dst2d = jnp.concatenate(
        [edge_index[1], N + (pad_iota % 8)]).reshape(NW, RT, CH)
    zeros = jnp.zeros((NH, D), jnp.float32)

    deg2 = _deg_kernel(dst2d).reshape(NC, NP)

    hs1 = pl.pallas_call(
        _mm1_body,
        out_shape=jax.ShapeDtypeStruct((NH, D), jnp.float32),
    )(x, W1, deg2)

    acc1 = _scatter_kernel(hs1, src2d, dst2d, zeros)

    hs2 = pl.pallas_call(
        _post1_body,
        out_shape=jax.ShapeDtypeStruct((NH, D), jnp.float32),
    )(acc1, deg2, b1.reshape(1, D), gamma1.reshape(1, D),
      beta1.reshape(1, D), W2)

    acc2 = _scatter_kernel(hs2, src2d, dst2d, zeros)

    out = pl.pallas_call(
        _post2_body,
        out_shape=jax.ShapeDtypeStruct((N, D), jnp.float32),
    )(acc2, deg2, b2.reshape(1, D), gamma2.reshape(1, D),
      beta2.reshape(1, D))
    return out


# speedup vs baseline: 33.8835x; 1.3383x over previous
import functools

import jax
import jax.numpy as jnp
from jax import lax
from jax.experimental import pallas as pl
from jax.experimental.pallas import tpu as pltpu
from jax.experimental.pallas import tpu_sc as plsc

N = 10000
D = 128
E = 320000
EPS = 1e-5
NP = 10240
NC = 2
NS = 16
NW = NC * NS
CH = 128
GR = 32
RT = 80
E_PAD = NW * RT * CH
PAD = E_PAD - E
NH = 10112
RPT = NH // NS
DPT = NP // NS

_mesh = plsc.VectorSubcoreMesh(core_axis_name="c", subcore_axis_name="s")


@functools.partial(
    pl.kernel,
    out_type=jax.ShapeDtypeStruct((NC * NP,), jnp.float32),
    mesh=_mesh,
    scratch_types=[
        pltpu.VMEM((RT, CH), jnp.int32),
        pltpu.VMEM((CH,), jnp.float32),
        pltpu.VMEM((DPT,), jnp.float32),
        pltpu.VMEM_SHARED((NP,), jnp.float32),
    ],
)
def _deg_kernel(dst_hbm, out_hbm, dst_v, ones, zbuf, dacc):
    c = lax.axis_index("c")
    s = lax.axis_index("s")
    wid = s * NC + c

    def fill_ones(i, carry):
        ones[pl.ds(i * 16, 16)] = jnp.full((16,), 1.0, jnp.float32)
        return carry

    lax.fori_loop(0, CH // 16, fill_ones, 0)

    def fill_z(i, carry):
        zbuf[pl.ds(i * 16, 16)] = jnp.zeros((16,), jnp.float32)
        return carry

    lax.fori_loop(0, DPT // 16, fill_z, 0)

    pltpu.sync_copy(zbuf, dacc.at[pl.ds(s * DPT, DPT)])
    pltpu.sync_copy(dst_hbm.at[wid], dst_v)
    plsc.subcore_barrier()

    def body(j, carry):
        pltpu.sync_copy(ones, dacc.at[dst_v.at[j]], add=True)
        return carry

    lax.fori_loop(0, RT, body, 0)
    plsc.subcore_barrier()
    pltpu.sync_copy(dacc.at[pl.ds(s * DPT, DPT)],
                    out_hbm.at[pl.ds(c * NP + s * DPT, DPT)])


@functools.partial(
    pl.kernel,
    out_type=jax.ShapeDtypeStruct((NC * NH, D), jnp.float32),
    mesh=_mesh,
    scratch_types=[
        pltpu.VMEM((RT, CH), jnp.int32),
        pltpu.VMEM((RT, CH), jnp.int32),
        pltpu.VMEM((GR, D), jnp.float32),
        pltpu.VMEM((GR, D), jnp.float32),
        pltpu.VMEM((GR, D), jnp.float32),
        pltpu.VMEM((GR, D), jnp.float32),
        pltpu.VMEM_SHARED((NH, D), jnp.float32),
        pltpu.SemaphoreType.DMA,
    ],
)
def _scatter_kernel(hs_hbm, src_hbm, dst_hbm, zeros_hbm, out_hbm,
                    src_v, dst_v, g0, g1, g2, g3, acc, sem0):
    c = lax.axis_index("c")
    s = lax.axis_index("s")
    wid = s * NC + c

    pltpu.sync_copy(src_hbm.at[wid], src_v)
    pltpu.sync_copy(dst_hbm.at[wid], dst_v)

    rows = pl.ds(s * RPT, RPT)

    @pl.when(c == 0)
    def _():
        pltpu.sync_copy(hs_hbm.at[rows], acc.at[rows])

    @pl.when(c == 1)
    def _():
        pltpu.sync_copy(zeros_hbm.at[rows], acc.at[rows])

    plsc.subcore_barrier()

    gs = (g0, g1, g2, g3)

    def fire(j, b, gbuf):
        return pltpu.async_copy(
            hs_hbm.at[src_v.at[j, pl.ds(b * GR, GR)]], gbuf, sem0)

    for b, gbuf in enumerate(gs):
        fire(0, b, gbuf)

    def body(j, carry):
        jn = jnp.minimum(j + 1, RT - 1)
        for b, gbuf in enumerate(gs):
            pltpu.make_async_copy(hs_hbm.at[pl.ds(0, GR)], gbuf, sem0).wait()
            pltpu.sync_copy(gbuf, acc.at[dst_v.at[j, pl.ds(b * GR, GR)]],
                            add=True)
            fire(jn, b, gbuf)
        return carry

    lax.fori_loop(0, RT, body, 0)
    for gbuf in gs:
        pltpu.make_async_copy(hs_hbm.at[pl.ds(0, GR)], gbuf, sem0).wait()

    plsc.subcore_barrier()
    pltpu.sync_copy(acc.at[rows], out_hbm.at[pl.ds(c * NH + s * RPT, RPT)])


def _dinv_col(deg_ref):
    dsum = deg_ref[0:1, :] + deg_ref[1:2, :] + 1.0
    return jnp.transpose(lax.rsqrt(dsum))[:N, :]


def _mm1_body(x_ref, w_ref, deg_ref, o_ref):
    dcol = _dinv_col(deg_ref)
    xs = x_ref[...] * dcol
    o_ref[0:N, :] = jnp.dot(xs, w_ref[...], preferred_element_type=jnp.float32)
    o_ref[N:NH, :] = jnp.zeros((NH - N, D), jnp.float32)


def _bn(agg, g_ref, be_ref):
    mu = jnp.mean(agg, axis=0, keepdims=True)
    var = jnp.mean(agg * agg, axis=0, keepdims=True) - mu * mu
    return (agg - mu) * lax.rsqrt(var + EPS) * g_ref[...] + be_ref[...]


def _post1_body(acc_ref, deg_ref, b_ref, g_ref, be_ref, w2_ref, o_ref):
    dcol = _dinv_col(deg_ref)
    agg = (acc_ref[0:N, :] + acc_ref[NH:NH + N, :]) * dcol + b_ref[...]
    y = jnp.maximum(_bn(agg, g_ref, be_ref), 0.0)
    h2 = jnp.dot(y, w2_ref[...], preferred_element_type=jnp.float32)
    o_ref[0:N, :] = h2 * dcol
    o_ref[N:NH, :] = jnp.zeros((NH - N, D), jnp.float32)


def _post2_body(acc_ref, deg_ref, b_ref, g_ref, be_ref, o_ref):
    dcol = _dinv_col(deg_ref)
    agg = (acc_ref[0:N, :] + acc_ref[NH:NH + N, :]) * dcol + b_ref[...]
    o_ref[...] = _bn(agg, g_ref, be_ref)


def kernel(x, edge_index, W1, b1, gamma1, beta1, W2, b2, gamma2, beta2):
    pad_iota = jnp.arange(PAD, dtype=jnp.int32)
    src2d = jnp.concatenate(
        [edge_index[0], pad_iota % N]).reshape(NW, RT, CH)
    dst2d = jnp.concatenate(
        [edge_index[1], N + (pad_iota % 8)]).reshape(NW, RT, CH)
    zeros = jnp.zeros((NH, D), jnp.float32)

    deg2 = _deg_kernel(dst2d).reshape(NC, NP)

    hs1 = pl.pallas_call(
        _mm1_body,
        out_shape=jax.ShapeDtypeStruct((NH, D), jnp.float32),
    )(x, W1, deg2)

    acc1 = _scatter_kernel(hs1, src2d, dst2d, zeros)

    hs2 = pl.pallas_call(
        _post1_body,
        out_shape=jax.ShapeDtypeStruct((NH, D), jnp.float32),
    )(acc1, deg2, b1.reshape(1, D), gamma1.reshape(1, D),
      beta1.reshape(1, D), W2)

    acc2 = _scatter_kernel(hs2, src2d, dst2d, zeros)

    out = pl.pallas_call(
        _post2_body,
        out_shape=jax.ShapeDtypeStruct((N, D), jnp.float32),
    )(acc2, deg2, b2.reshape(1, D), gamma2.reshape(1, D),
      beta2.reshape(1, D))
    return out


# speedup vs baseline: 33.9180x; 1.0010x over previous
import functools

import jax
import jax.numpy as jnp
from jax import lax
from jax.experimental import pallas as pl
from jax.experimental.pallas import tpu as pltpu
from jax.experimental.pallas import tpu_sc as plsc

N = 10000
D = 128
E = 320000
EPS = 1e-5
NP = 10240
NC = 2
NS = 16
NW = NC * NS
CH = 128
GR = 16
RT = 80
E_PAD = NW * RT * CH
PAD = E_PAD - E
NH = 10112
RPT = NH // NS
DPT = NP // NS

_mesh = plsc.VectorSubcoreMesh(core_axis_name="c", subcore_axis_name="s")


@functools.partial(
    pl.kernel,
    out_type=jax.ShapeDtypeStruct((NC * NP,), jnp.float32),
    mesh=_mesh,
    scratch_types=[
        pltpu.VMEM((RT, CH), jnp.int32),
        pltpu.VMEM((CH,), jnp.float32),
        pltpu.VMEM((DPT,), jnp.float32),
        pltpu.VMEM_SHARED((NP,), jnp.float32),
    ],
)
def _deg_kernel(dst_hbm, out_hbm, dst_v, ones, zbuf, dacc):
    c = lax.axis_index("c")
    s = lax.axis_index("s")
    wid = s * NC + c

    def fill_ones(i, carry):
        ones[pl.ds(i * 16, 16)] = jnp.full((16,), 1.0, jnp.float32)
        return carry

    lax.fori_loop(0, CH // 16, fill_ones, 0)

    def fill_z(i, carry):
        zbuf[pl.ds(i * 16, 16)] = jnp.zeros((16,), jnp.float32)
        return carry

    lax.fori_loop(0, DPT // 16, fill_z, 0)

    pltpu.sync_copy(zbuf, dacc.at[pl.ds(s * DPT, DPT)])
    pltpu.sync_copy(dst_hbm.at[wid], dst_v)
    plsc.subcore_barrier()

    def body(j, carry):
        pltpu.sync_copy(ones, dacc.at[dst_v.at[j]], add=True)
        return carry

    lax.fori_loop(0, RT, body, 0)
    plsc.subcore_barrier()
    pltpu.sync_copy(dacc.at[pl.ds(s * DPT, DPT)],
                    out_hbm.at[pl.ds(c * NP + s * DPT, DPT)])


@functools.partial(
    pl.kernel,
    out_type=jax.ShapeDtypeStruct((NC * NH, D), jnp.float32),
    mesh=_mesh,
    scratch_types=[
        pltpu.VMEM((RT, CH), jnp.int32),
        pltpu.VMEM((RT, CH), jnp.int32),
        pltpu.VMEM((GR, D), jnp.float32),
        pltpu.VMEM((GR, D), jnp.float32),
        pltpu.VMEM((GR, D), jnp.float32),
        pltpu.VMEM((GR, D), jnp.float32),
        pltpu.VMEM((GR, D), jnp.float32),
        pltpu.VMEM((GR, D), jnp.float32),
        pltpu.VMEM((GR, D), jnp.float32),
        pltpu.VMEM((GR, D), jnp.float32),
        pltpu.VMEM_SHARED((NH, D), jnp.float32),
        pltpu.SemaphoreType.DMA,
    ],
)
def _scatter_kernel(hs_hbm, src_hbm, dst_hbm, zeros_hbm, out_hbm,
                    src_v, dst_v, g0, g1, g2, g3, g4, g5, g6, g7, acc, sem0):
    c = lax.axis_index("c")
    s = lax.axis_index("s")
    wid = s * NC + c

    pltpu.sync_copy(src_hbm.at[wid], src_v)
    pltpu.sync_copy(dst_hbm.at[wid], dst_v)

    rows = pl.ds(s * RPT, RPT)

    @pl.when(c == 0)
    def _():
        pltpu.sync_copy(hs_hbm.at[rows], acc.at[rows])

    @pl.when(c == 1)
    def _():
        pltpu.sync_copy(zeros_hbm.at[rows], acc.at[rows])

    plsc.subcore_barrier()

    gs = (g0, g1, g2, g3, g4, g5, g6, g7)

    def fire(j, b, gbuf):
        return pltpu.async_copy(
            hs_hbm.at[src_v.at[j, pl.ds(b * GR, GR)]], gbuf, sem0)

    for b, gbuf in enumerate(gs):
        fire(0, b, gbuf)

    def body(j, carry):
        jn = jnp.minimum(j + 1, RT - 1)
        for b, gbuf in enumerate(gs):
            pltpu.make_async_copy(hs_hbm.at[pl.ds(0, GR)], gbuf, sem0).wait()
            pltpu.sync_copy(gbuf, acc.at[dst_v.at[j, pl.ds(b * GR, GR)]],
                            add=True)
            fire(jn, b, gbuf)
        return carry

    lax.fori_loop(0, RT, body, 0)
    for gbuf in gs:
        pltpu.make_async_copy(hs_hbm.at[pl.ds(0, GR)], gbuf, sem0).wait()

    plsc.subcore_barrier()
    pltpu.sync_copy(acc.at[rows], out_hbm.at[pl.ds(c * NH + s * RPT, RPT)])


def _dinv_col(deg_ref):
    dsum = deg_ref[0:1, :] + deg_ref[1:2, :] + 1.0
    return jnp.transpose(lax.rsqrt(dsum))[:N, :]


def _mm1_body(x_ref, w_ref, deg_ref, o_ref):
    dcol = _dinv_col(deg_ref)
    xs = x_ref[...] * dcol
    o_ref[0:N, :] = jnp.dot(xs, w_ref[...], preferred_element_type=jnp.float32)
    o_ref[N:NH, :] = jnp.zeros((NH - N, D), jnp.float32)


def _bn(agg, g_ref, be_ref):
    mu = jnp.mean(agg, axis=0, keepdims=True)
    var = jnp.mean(agg * agg, axis=0, keepdims=True) - mu * mu
    return (agg - mu) * lax.rsqrt(var + EPS) * g_ref[...] + be_ref[...]


def _post1_body(acc_ref, deg_ref, b_ref, g_ref, be_ref, w2_ref, o_ref):
    dcol = _dinv_col(deg_ref)
    agg = (acc_ref[0:N, :] + acc_ref[NH:NH + N, :]) * dcol + b_ref[...]
    y = jnp.maximum(_bn(agg, g_ref, be_ref), 0.0)
    h2 = jnp.dot(y, w2_ref[...], preferred_element_type=jnp.float32)
    o_ref[0:N, :] = h2 * dcol
    o_ref[N:NH, :] = jnp.zeros((NH - N, D), jnp.float32)


def _post2_body(acc_ref, deg_ref, b_ref, g_ref, be_ref, o_ref):
    dcol = _dinv_col(deg_ref)
    agg = (acc_ref[0:N, :] + acc_ref[NH:NH + N, :]) * dcol + b_ref[...]
    o_ref[...] = _bn(agg, g_ref, be_ref)


def kernel(x, edge_index, W1, b1, gamma1, beta1, W2, b2, gamma2, beta2):
    pad_iota = jnp.arange(PAD, dtype=jnp.int32)
    src2d = jnp.concatenate(
        [edge_index[0], pad_iota % N]).reshape(NW, RT, CH)
    dst2d = jnp.concatenate(
        [edge_index[1], N + (pad_iota % 8)]).reshape(NW, RT, CH)
    zeros = jnp.zeros((NH, D), jnp.float32)

    deg2 = _deg_kernel(dst2d).reshape(NC, NP)

    hs1 = pl.pallas_call(
        _mm1_body,
        out_shape=jax.ShapeDtypeStruct((NH, D), jnp.float32),
    )(x, W1, deg2)

    acc1 = _scatter_kernel(hs1, src2d, dst2d, zeros)

    hs2 = pl.pallas_call(
        _post1_body,
        out_shape=jax.ShapeDtypeStruct((NH, D), jnp.float32),
    )(acc1, deg2, b1.reshape(1, D), gamma1.reshape(1, D),
      beta1.reshape(1, D), W2)

    acc2 = _scatter_kernel(hs2, src2d, dst2d, zeros)

    out = pl.pallas_call(
        _post2_body,
        out_shape=jax.ShapeDtypeStruct((N, D), jnp.float32),
    )(acc2, deg2, b2.reshape(1, D), gamma2.reshape(1, D),
      beta2.reshape(1, D))
    return out
